# Initial kernel scaffold; baseline (speedup 1.0000x reference)
import functools

import jax
import jax.numpy as jnp
from jax import lax
from jax.experimental import pallas as pl
from jax.experimental.pallas import tpu as pltpu
from jax.experimental.pallas import tpu_sc as plsc

N = 10000
NPAD = 10240
E = 320000
D_IN = 128
D_HID = 128
D_OUT = 40
DP3 = 48

NC, NS = 2, 16
NW = NC * NS
EPW = E // NW
CH = 80
NCHUNK = EPW // CH
RPT = NPAD // NS

_mesh = lambda: plsc.VectorSubcoreMesh(core_axis_name="c", subcore_axis_name="s")


@functools.partial(
    pl.kernel,
    out_type=jax.ShapeDtypeStruct((NC, NPAD), jnp.float32),
    mesh=_mesh(),
    scratch_types=[
        pltpu.VMEM((CH,), jnp.int32),
        pltpu.VMEM((CH,), jnp.float32),
        pltpu.VMEM((RPT,), jnp.float32),
        pltpu.VMEM_SHARED((NPAD,), jnp.float32),
    ],
)
def _deg_kernel(dst_hbm, ew_hbm, out_hbm, dstv, eww, zbuf, acc):
    cid = lax.axis_index("c")
    sid = lax.axis_index("s")
    wid = cid * NS + sid

    def zb(i, _):
        zbuf[pl.ds(i * 16, 16)] = jnp.zeros((16,), jnp.float32)
        return 0
    lax.fori_loop(0, RPT // 16, zb, 0)
    pltpu.sync_copy(zbuf, acc.at[pl.ds(sid * RPT, RPT)])
    plsc.subcore_barrier()

    def chunk(k, _):
        base = wid * EPW + k * CH
        pltpu.sync_copy(dst_hbm.at[pl.ds(base, CH)], dstv)
        pltpu.sync_copy(ew_hbm.at[pl.ds(base, CH)], eww)
        pltpu.sync_copy(eww, acc.at[dstv], add=True)
        return 0
    lax.fori_loop(0, NCHUNK, chunk, 0)
    plsc.subcore_barrier()
    pltpu.sync_copy(acc.at[pl.ds(sid * RPT, RPT)],
                    out_hbm.at[cid, pl.ds(sid * RPT, RPT)])


def _make_agg(D):
    grp = D // 16

    @functools.partial(
        pl.kernel,
        out_type=jax.ShapeDtypeStruct((NC, NPAD, D), jnp.float32),
        mesh=_mesh(),
        scratch_types=[
            pltpu.VMEM((CH,), jnp.int32),
            pltpu.VMEM((CH,), jnp.int32),
            pltpu.VMEM((CH,), jnp.float32),
            pltpu.VMEM((CH, D), jnp.float32),
            pltpu.VMEM((128, D), jnp.float32),
            pltpu.VMEM_SHARED((NPAD, D), jnp.float32),
            pltpu.SemaphoreType.DMA,
        ],
    )
    def agg(g_hbm, src_hbm, dst_hbm, ew_hbm, out_hbm,
            srcv, dstv, eww, rows, zbuf, acc, sem):
        cid = lax.axis_index("c")
        sid = lax.axis_index("s")
        wid = cid * NS + sid

        def zb(i, _):
            for c in range(grp):
                zbuf[i, pl.ds(c * 16, 16)] = jnp.zeros((16,), jnp.float32)
            return 0
        lax.fori_loop(0, 128, zb, 0)
        for j in range(RPT // 128):
            pltpu.sync_copy(zbuf, acc.at[pl.ds(sid * RPT + j * 128, 128)])
        plsc.subcore_barrier()

        def chunk(k, _):
            base = wid * EPW + k * CH
            pltpu.sync_copy(src_hbm.at[pl.ds(base, CH)], srcv)
            pltpu.sync_copy(dst_hbm.at[pl.ds(base, CH)], dstv)
            pltpu.sync_copy(ew_hbm.at[pl.ds(base, CH)], eww)
            pltpu.async_copy(g_hbm.at[srcv], rows, sem).wait()

            def scale16(g2, _):
                for jj in range(16):
                    r = g2 * 16 + jj
                    ewb = plsc.load_gather(
                        eww, [jnp.full((16,), r, jnp.int32)])
                    for c in range(grp):
                        sl = pl.ds(c * 16, 16)
                        rows[r, sl] = rows[r, sl] * ewb
                return 0
            lax.fori_loop(0, CH // 16, scale16, 0)
            pltpu.sync_copy(rows, acc.at[dstv], add=True)
            return 0
        lax.fori_loop(0, NCHUNK, chunk, 0)
        plsc.subcore_barrier()
        pltpu.sync_copy(acc.at[pl.ds(sid * RPT, RPT)],
                        out_hbm.at[cid, pl.ds(sid * RPT, RPT)])

    return agg


_agg128 = _make_agg(D_HID)
_agg48 = _make_agg(DP3)


_R = 1024
_G = NPAD // _R


def _tc1_body(degA_ref, degB_ref, x_ref, w_ref, g_ref, dis_ref):
    deg = degA_ref[...] + degB_ref[...] + 1.0
    dis = lax.rsqrt(deg)
    z = jnp.dot(x_ref[...], w_ref[...], preferred_element_type=jnp.float32)
    g_ref[...] = z * dis
    dis_ref[...] = dis


def _tc1(degA, degB, x, W1):
    return pl.pallas_call(
        _tc1_body,
        grid=(_G,),
        in_specs=[
            pl.BlockSpec((_R, 1), lambda i: (i, 0)),
            pl.BlockSpec((_R, 1), lambda i: (i, 0)),
            pl.BlockSpec((_R, D_IN), lambda i: (i, 0)),
            pl.BlockSpec((D_IN, D_HID), lambda i: (0, 0)),
        ],
        out_specs=[
            pl.BlockSpec((_R, D_HID), lambda i: (i, 0)),
            pl.BlockSpec((_R, 1), lambda i: (i, 0)),
        ],
        out_shape=[
            jax.ShapeDtypeStruct((NPAD, D_HID), jnp.float32),
            jax.ShapeDtypeStruct((NPAD, 1), jnp.float32),
        ],
    )(degA, degB, x, W1)


def _make_tcmid(Dp, Dn):
    def body(pa_ref, pb_ref, gp_ref, dis_ref, b_ref, w_ref, g_ref):
        dis = dis_ref[...]
        y = jax.nn.relu((pa_ref[...] + pb_ref[...] + gp_ref[...]) * dis
                        + b_ref[...])
        z = jnp.dot(y, w_ref[...], preferred_element_type=jnp.float32)
        g_ref[...] = z * dis

    def call(pa, pb, gp, dis, b, W):
        return pl.pallas_call(
            body,
            grid=(_G,),
            in_specs=[
                pl.BlockSpec((_R, Dp), lambda i: (i, 0)),
                pl.BlockSpec((_R, Dp), lambda i: (i, 0)),
                pl.BlockSpec((_R, Dp), lambda i: (i, 0)),
                pl.BlockSpec((_R, 1), lambda i: (i, 0)),
                pl.BlockSpec((1, Dp), lambda i: (0, 0)),
                pl.BlockSpec((Dp, Dn), lambda i: (0, 0)),
            ],
            out_specs=pl.BlockSpec((_R, Dn), lambda i: (i, 0)),
            out_shape=jax.ShapeDtypeStruct((NPAD, Dn), jnp.float32),
        )(pa, pb, gp, dis, b, W)

    return call


_tcmid_128 = _make_tcmid(D_HID, D_HID)
_tcmid_48 = _make_tcmid(D_HID, DP3)


def _tcfinal_body(pa_ref, pb_ref, gp_ref, dis_ref, b_ref, o_ref):
    dis = dis_ref[...]
    y = jax.nn.relu((pa_ref[...] + pb_ref[...] + gp_ref[...]) * dis
                    + b_ref[...])
    col = lax.broadcasted_iota(jnp.int32, (_R, DP3), 1)
    y = jnp.where(col < D_OUT, y, -jnp.inf)
    m = jnp.max(y, axis=1, keepdims=True)
    ex = jnp.exp(y - m)
    s = jnp.sum(ex, axis=1, keepdims=True)
    o_ref[...] = y - m - jnp.log(s)


def _tcfinal(pa, pb, gp, dis, b):
    return pl.pallas_call(
        _tcfinal_body,
        grid=(_G,),
        in_specs=[
            pl.BlockSpec((_R, DP3), lambda i: (i, 0)),
            pl.BlockSpec((_R, DP3), lambda i: (i, 0)),
            pl.BlockSpec((_R, DP3), lambda i: (i, 0)),
            pl.BlockSpec((_R, 1), lambda i: (i, 0)),
            pl.BlockSpec((1, DP3), lambda i: (0, 0)),
        ],
        out_specs=pl.BlockSpec((_R, DP3), lambda i: (i, 0)),
        out_shape=jax.ShapeDtypeStruct((NPAD, DP3), jnp.float32),
    )(pa, pb, gp, dis, b)


def kernel(x, edge_index, edge_attr, W1, b1, W2, b2, W3, b3):
    src = edge_index[0].astype(jnp.int32)
    dst = edge_index[1].astype(jnp.int32)
    ew = edge_attr.astype(jnp.float32)
    xp = jnp.zeros((NPAD, D_IN), jnp.float32).at[:N].set(x)
    W3p = jnp.pad(W3, ((0, 0), (0, DP3 - D_OUT)))
    b3p = jnp.pad(b3, (0, DP3 - D_OUT)).reshape(1, DP3)

    degp = _deg_kernel(dst, ew)
    degA = degp[0].reshape(NPAD, 1)
    degB = degp[1].reshape(NPAD, 1)

    g1, dis = _tc1(degA, degB, xp, W1)
    p = _agg128(g1, src, dst, ew)
    g2 = _tcmid_128(p[0], p[1], g1, dis, b1.reshape(1, D_HID), W2)
    p = _agg128(g2, src, dst, ew)
    g3 = _tcmid_48(p[0], p[1], g2, dis, b2.reshape(1, D_HID), W3p)
    p = _agg48(g3, src, dst, ew)
    y = _tcfinal(p[0], p[1], g3, dis, b3p)
    return y[:N, :D_OUT]


# speedup vs baseline: 7.5997x; 7.5997x over previous
import functools

import jax
import jax.numpy as jnp
from jax import lax
from jax.experimental import pallas as pl
from jax.experimental.pallas import tpu as pltpu
from jax.experimental.pallas import tpu_sc as plsc

N = 10000
NPAD = 10240
E = 320000
D_IN = 128
D_HID = 128
D_OUT = 40
DP3 = 128

NC, NS = 2, 16
NW = NC * NS
EPW = E // NW
CH = 80
NCHUNK = EPW // CH
RPT = NPAD // NS

_mesh = lambda: plsc.VectorSubcoreMesh(core_axis_name="c", subcore_axis_name="s")


@functools.partial(
    pl.kernel,
    out_type=jax.ShapeDtypeStruct((NC, NPAD), jnp.float32),
    mesh=_mesh(),
    scratch_types=[
        pltpu.VMEM((CH,), jnp.int32),
        pltpu.VMEM((CH,), jnp.float32),
        pltpu.VMEM((RPT,), jnp.float32),
        pltpu.VMEM_SHARED((NPAD,), jnp.float32),
    ],
    compiler_params=pltpu.CompilerParams(needs_layout_passes=False),
)
def _deg_kernel(dst_hbm, ew_hbm, out_hbm, dstv, eww, zbuf, acc):
    cid = lax.axis_index("c")
    sid = lax.axis_index("s")
    wid = cid * NS + sid

    def zb(i, _):
        zbuf[pl.ds(i * 16, 16)] = jnp.zeros((16,), jnp.float32)
        return 0
    lax.fori_loop(0, RPT // 16, zb, 0)
    pltpu.sync_copy(zbuf, acc.at[pl.ds(sid * RPT, RPT)])
    plsc.subcore_barrier()

    def chunk(k, _):
        base = wid * EPW + k * CH
        pltpu.sync_copy(dst_hbm.at[pl.ds(base, CH)], dstv)
        pltpu.sync_copy(ew_hbm.at[pl.ds(base, CH)], eww)
        pltpu.sync_copy(eww, acc.at[dstv], add=True)
        return 0
    lax.fori_loop(0, NCHUNK, chunk, 0)
    plsc.subcore_barrier()
    pltpu.sync_copy(acc.at[pl.ds(sid * RPT, RPT)],
                    out_hbm.at[cid, pl.ds(sid * RPT, RPT)])


def _make_agg(D):
    grp = D // 16

    @functools.partial(
        pl.kernel,
        out_type=jax.ShapeDtypeStruct((NC, NPAD, D), jnp.float32),
        mesh=_mesh(),
        scratch_types=[
            pltpu.VMEM((CH,), jnp.int32),
            pltpu.VMEM((CH,), jnp.int32),
            pltpu.VMEM((CH,), jnp.float32),
            pltpu.VMEM((CH, D), jnp.float32),
            pltpu.VMEM((128, D), jnp.float32),
            pltpu.VMEM_SHARED((NPAD, D), jnp.float32),
            pltpu.SemaphoreType.DMA,
        ],
        compiler_params=pltpu.CompilerParams(needs_layout_passes=False),
    )
    def agg(g_hbm, src_hbm, dst_hbm, ew_hbm, out_hbm,
            srcv, dstv, eww, rows, zbuf, acc, sem):
        cid = lax.axis_index("c")
        sid = lax.axis_index("s")
        wid = cid * NS + sid

        def zb(i, _):
            for c in range(grp):
                zbuf[i, pl.ds(c * 16, 16)] = jnp.zeros((16,), jnp.float32)
            return 0
        lax.fori_loop(0, 128, zb, 0)
        for j in range(RPT // 128):
            pltpu.sync_copy(zbuf, acc.at[pl.ds(sid * RPT + j * 128, 128)])
        plsc.subcore_barrier()

        def chunk(k, _):
            base = wid * EPW + k * CH
            pltpu.sync_copy(src_hbm.at[pl.ds(base, CH)], srcv)
            pltpu.sync_copy(dst_hbm.at[pl.ds(base, CH)], dstv)
            pltpu.sync_copy(ew_hbm.at[pl.ds(base, CH)], eww)
            pltpu.async_copy(g_hbm.at[srcv], rows, sem).wait()

            def scale16(g2, _):
                for jj in range(16):
                    r = g2 * 16 + jj
                    ewb = plsc.load_gather(
                        eww, [jnp.full((16,), r, jnp.int32)])
                    for c in range(grp):
                        sl = pl.ds(c * 16, 16)
                        rows[r, sl] = rows[r, sl] * ewb
                return 0
            lax.fori_loop(0, CH // 16, scale16, 0)
            pltpu.sync_copy(rows, acc.at[dstv], add=True)
            return 0
        lax.fori_loop(0, NCHUNK, chunk, 0)
        plsc.subcore_barrier()
        pltpu.sync_copy(acc.at[pl.ds(sid * RPT, RPT)],
                        out_hbm.at[cid, pl.ds(sid * RPT, RPT)])

    return agg


_agg128 = _make_agg(D_HID)


_R = 1024
_G = NPAD // _R


def _tc1_body(degA_ref, degB_ref, x_ref, w_ref, g_ref, dis_ref):
    deg = degA_ref[...] + degB_ref[...] + 1.0
    dis = lax.rsqrt(deg)
    z = jnp.dot(x_ref[...], w_ref[...], preferred_element_type=jnp.float32)
    g_ref[...] = z * dis
    dis_ref[...] = dis


def _tc1(degA, degB, x, W1):
    return pl.pallas_call(
        _tc1_body,
        grid=(_G,),
        in_specs=[
            pl.BlockSpec((_R, 1), lambda i: (i, 0)),
            pl.BlockSpec((_R, 1), lambda i: (i, 0)),
            pl.BlockSpec((_R, D_IN), lambda i: (i, 0)),
            pl.BlockSpec((D_IN, D_HID), lambda i: (0, 0)),
        ],
        out_specs=[
            pl.BlockSpec((_R, D_HID), lambda i: (i, 0)),
            pl.BlockSpec((_R, 1), lambda i: (i, 0)),
        ],
        out_shape=[
            jax.ShapeDtypeStruct((NPAD, D_HID), jnp.float32),
            jax.ShapeDtypeStruct((NPAD, 1), jnp.float32),
        ],
    )(degA, degB, x, W1)


def _make_tcmid(Dp, Dn):
    def body(pa_ref, pb_ref, gp_ref, dis_ref, b_ref, w_ref, g_ref):
        dis = dis_ref[...]
        y = jax.nn.relu((pa_ref[...] + pb_ref[...] + gp_ref[...]) * dis
                        + b_ref[...])
        z = jnp.dot(y, w_ref[...], preferred_element_type=jnp.float32)
        g_ref[...] = z * dis

    def call(pa, pb, gp, dis, b, W):
        return pl.pallas_call(
            body,
            grid=(_G,),
            in_specs=[
                pl.BlockSpec((_R, Dp), lambda i: (i, 0)),
                pl.BlockSpec((_R, Dp), lambda i: (i, 0)),
                pl.BlockSpec((_R, Dp), lambda i: (i, 0)),
                pl.BlockSpec((_R, 1), lambda i: (i, 0)),
                pl.BlockSpec((1, Dp), lambda i: (0, 0)),
                pl.BlockSpec((Dp, Dn), lambda i: (0, 0)),
            ],
            out_specs=pl.BlockSpec((_R, Dn), lambda i: (i, 0)),
            out_shape=jax.ShapeDtypeStruct((NPAD, Dn), jnp.float32),
        )(pa, pb, gp, dis, b, W)

    return call


_tcmid_128 = _make_tcmid(D_HID, D_HID)


def _tcfinal_body(pa_ref, pb_ref, gp_ref, dis_ref, b_ref, o_ref):
    dis = dis_ref[...]
    y = jax.nn.relu((pa_ref[...] + pb_ref[...] + gp_ref[...]) * dis
                    + b_ref[...])
    col = lax.broadcasted_iota(jnp.int32, (_R, DP3), 1)
    y = jnp.where(col < D_OUT, y, -jnp.inf)
    m = jnp.max(y, axis=1, keepdims=True)
    ex = jnp.exp(y - m)
    s = jnp.sum(ex, axis=1, keepdims=True)
    o_ref[...] = y - m - jnp.log(s)


def _tcfinal(pa, pb, gp, dis, b):
    return pl.pallas_call(
        _tcfinal_body,
        grid=(_G,),
        in_specs=[
            pl.BlockSpec((_R, DP3), lambda i: (i, 0)),
            pl.BlockSpec((_R, DP3), lambda i: (i, 0)),
            pl.BlockSpec((_R, DP3), lambda i: (i, 0)),
            pl.BlockSpec((_R, 1), lambda i: (i, 0)),
            pl.BlockSpec((1, DP3), lambda i: (0, 0)),
        ],
        out_specs=pl.BlockSpec((_R, DP3), lambda i: (i, 0)),
        out_shape=jax.ShapeDtypeStruct((NPAD, DP3), jnp.float32),
    )(pa, pb, gp, dis, b)


def kernel(x, edge_index, edge_attr, W1, b1, W2, b2, W3, b3):
    src = edge_index[0].astype(jnp.int32)
    dst = edge_index[1].astype(jnp.int32)
    ew = edge_attr.astype(jnp.float32)
    xp = jnp.zeros((NPAD, D_IN), jnp.float32).at[:N].set(x)
    W3p = jnp.pad(W3, ((0, 0), (0, DP3 - D_OUT)))
    b3p = jnp.pad(b3, (0, DP3 - D_OUT)).reshape(1, DP3)

    degp = _deg_kernel(dst, ew)
    degA = degp[0].reshape(NPAD, 1)
    degB = degp[1].reshape(NPAD, 1)

    g1, dis = _tc1(degA, degB, xp, W1)
    p = _agg128(g1, src, dst, ew)
    g2 = _tcmid_128(p[0], p[1], g1, dis, b1.reshape(1, D_HID), W2)
    p = _agg128(g2, src, dst, ew)
    g3 = _tcmid_128(p[0], p[1], g2, dis, b2.reshape(1, D_HID), W3p)
    p = _agg128(g3, src, dst, ew)
    y = _tcfinal(p[0], p[1], g3, dis, b3p)
    return y[:N, :D_OUT]
